# Initial kernel scaffold; baseline (speedup 1.0000x reference)
import jax
import jax.numpy as jnp
from jax.experimental import pallas as pl


def kernel(x, edge_index, batch, W1, b1, W2, b2, W3, b3, Wc1, bc1, Wc2, bc2):
    raise NotImplementedError("write your pallas kernel here")


# speedup vs baseline: 16.7751x; 16.7751x over previous
import functools

import jax
import jax.numpy as jnp
from jax import lax
from jax.experimental import pallas as pl
from jax.experimental.pallas import tpu as pltpu
from jax.experimental.pallas import tpu_sc as plsc

N = 10000
E = 320000
F = 128
H = 64
G = 128
C = 10

NC = 2
NS = 16
NW = NC * NS
CH = 128
NCHUNK = E // CH
FULL = NCHUNK // NW
REM = NCHUNK - FULL * NW

_mesh = lambda: plsc.VectorSubcoreMesh(core_axis_name="c", subcore_axis_name="s")
_SC_PARAMS = pltpu.CompilerParams(use_tc_tiling_on_sc=False)


def _wid():
    return lax.axis_index("c") * NS + lax.axis_index("s")


def _deg_body(dst_hbm, zeros_hbm, ones_hbm, degp_hbm, dstv, onesv, deg_sh):
    cid = lax.axis_index("c")
    sid = lax.axis_index("s")
    wid = cid * NS + sid

    @pl.when(sid == 0)
    def _():
        pltpu.sync_copy(zeros_hbm, deg_sh)

    pltpu.sync_copy(ones_hbm, onesv)
    plsc.subcore_barrier()

    def do_chunk(ch):
        pltpu.sync_copy(dst_hbm.at[pl.ds(ch * CH, CH)], dstv)
        pltpu.sync_copy(onesv, deg_sh.at[dstv], add=True)

    def step(i, carry):
        do_chunk(i * NW + wid)
        return carry

    lax.fori_loop(0, FULL, step, 0)

    @pl.when(wid < REM)
    def _():
        do_chunk(FULL * NW + wid)

    plsc.subcore_barrier()

    @pl.when(sid == 0)
    def _():
        pltpu.sync_copy(deg_sh, degp_hbm.at[cid])


def _deg_call(dst, zeros1, ones1):
    return pl.kernel(
        _deg_body,
        out_type=jax.ShapeDtypeStruct((NC, N), jnp.float32),
        mesh=_mesh(),
        compiler_params=_SC_PARAMS,
        scratch_types=[
            pltpu.VMEM((CH,), jnp.int32),
            pltpu.VMEM((CH,), jnp.float32),
            pltpu.VMEM_SHARED((N,), jnp.float32),
        ],
    )(dst, zeros1, ones1)


def _agg_body(hs_hbm, src_hbm, dst_hbm, zeros_hbm, out_hbm, srcv, dstv, rows, acc_sh):
    cid = lax.axis_index("c")
    sid = lax.axis_index("s")
    wid = cid * NS + sid

    @pl.when(sid == 0)
    def _():
        pltpu.sync_copy(zeros_hbm, acc_sh)

    plsc.subcore_barrier()

    def do_chunk(ch):
        base = ch * CH
        pltpu.sync_copy(src_hbm.at[pl.ds(base, CH)], srcv)
        pltpu.sync_copy(dst_hbm.at[pl.ds(base, CH)], dstv)
        pltpu.sync_copy(hs_hbm.at[srcv], rows)
        pltpu.sync_copy(rows, acc_sh.at[dstv], add=True)

    def step(i, carry):
        do_chunk(i * NW + wid)
        return carry

    lax.fori_loop(0, FULL, step, 0)

    @pl.when(wid < REM)
    def _():
        do_chunk(FULL * NW + wid)

    plsc.subcore_barrier()

    @pl.when(sid == 0)
    def _():
        pltpu.sync_copy(acc_sh, out_hbm.at[cid])


def _agg_call(hs, src, dst, zeros2):
    return pl.kernel(
        _agg_body,
        out_type=jax.ShapeDtypeStruct((NC, N, H), jnp.float32),
        mesh=_mesh(),
        compiler_params=_SC_PARAMS,
        scratch_types=[
            pltpu.VMEM((CH,), jnp.int32),
            pltpu.VMEM((CH,), jnp.int32),
            pltpu.VMEM((CH, H), jnp.float32),
            pltpu.VMEM_SHARED((N, H), jnp.float32),
        ],
    )(hs, src, dst, zeros2)


def _prep_body(degpt_ref, x_ref, w1_ref, dis2d_ref, hs1_ref):
    s = degpt_ref[...]
    deg = s[:, 0:1] + s[:, 1:2] + 1.0
    dis = lax.rsqrt(deg)
    d2 = jnp.broadcast_to(dis, (N, H))
    dis2d_ref[...] = d2
    mm = jnp.dot(x_ref[...], w1_ref[...], preferred_element_type=jnp.float32)
    hs1_ref[...] = d2 * mm


def _prep_call(degp_t, x, w1):
    return pl.pallas_call(
        _prep_body,
        out_shape=(
            jax.ShapeDtypeStruct((N, H), jnp.float32),
            jax.ShapeDtypeStruct((N, H), jnp.float32),
        ),
    )(degp_t, x, w1)


def _mid_body(pp_ref, hs_ref, d2_ref, w_ref, b_ref, out_ref):
    p = jnp.sum(pp_ref[...], axis=0)
    agg = p + hs_ref[...]
    d2 = d2_ref[...]
    h = jnp.maximum(d2 * agg + b_ref[...], 0.0)
    out_ref[...] = d2 * jnp.dot(h, w_ref[...], preferred_element_type=jnp.float32)


def _mid_call(pp, hs, dis2d, w_next, b):
    return pl.pallas_call(
        _mid_body,
        out_shape=jax.ShapeDtypeStruct((N, H), jnp.float32),
    )(pp, hs, dis2d, w_next, b)


def _final_body(pp_ref, hs_ref, d2_ref, b3_ref, batch_ref, wc1_ref, bc1_ref,
                wc2_ref, bc2_ref, out_ref):
    p = jnp.sum(pp_ref[...], axis=0)
    agg = p + hs_ref[...]
    h = jnp.maximum(d2_ref[...] * agg + b3_ref[...], 0.0)
    bb = batch_ref[...]
    gi = lax.broadcasted_iota(jnp.int32, (G, N), 0)
    oh = jnp.where(gi == bb, 1.0, 0.0)
    sums = jnp.dot(oh, h, preferred_element_type=jnp.float32)
    cnts = jnp.sum(oh, axis=1, keepdims=True)
    g = sums / jnp.maximum(cnts, 1.0)
    a = jnp.maximum(
        jnp.dot(g, wc1_ref[...], preferred_element_type=jnp.float32) + bc1_ref[...],
        0.0,
    )
    out_ref[...] = (
        jnp.dot(a, wc2_ref[...], preferred_element_type=jnp.float32) + bc2_ref[...]
    )


def _final_call(pp, hs, dis2d, b3, batch2d, wc1, bc1, wc2, bc2):
    return pl.pallas_call(
        _final_body,
        out_shape=jax.ShapeDtypeStruct((G, C), jnp.float32),
    )(pp, hs, dis2d, b3, batch2d, wc1, bc1, wc2, bc2)


@jax.jit
def kernel(x, edge_index, batch, W1, b1, W2, b2, W3, b3, Wc1, bc1, Wc2, bc2):
    src = edge_index[0]
    dst = edge_index[1]
    zeros1 = jnp.zeros((N,), jnp.float32)
    ones1 = jnp.ones((CH,), jnp.float32)
    zeros2 = jnp.zeros((N, H), jnp.float32)

    degp = _deg_call(dst, zeros1, ones1)
    dis2d, hs = _prep_call(degp.T, x, W1)

    for (w_next, b) in ((W2, b1), (W3, b2)):
        pp = _agg_call(hs, src, dst, zeros2)
        hs = _mid_call(pp, hs, dis2d, w_next, b.reshape(1, H))

    pp = _agg_call(hs, src, dst, zeros2)
    out = _final_call(
        pp, hs, dis2d, b3.reshape(1, H), batch.reshape(1, N).astype(jnp.int32),
        Wc1, bc1.reshape(1, H // 2), Wc2, bc2.reshape(1, C),
    )
    return out


# speedup vs baseline: 20.8575x; 1.2434x over previous
import jax
import jax.numpy as jnp
from jax import lax
from jax.experimental import pallas as pl
from jax.experimental.pallas import tpu as pltpu
from jax.experimental.pallas import tpu_sc as plsc

N = 10000
E = 320000
F = 128
H = 64
HH = H // 2
G = 128
C = 10

NC = 2
NS = 16
NW = NC * NS
CH = 128

CPT = 160
K = 5
T = CPT // K
NCHUNK = NS * CPT
E2 = NCHUNK * CH
NPAD = 8
N2 = N + NPAD

_mesh = lambda: plsc.VectorSubcoreMesh(core_axis_name="c", subcore_axis_name="s")
_SC_PARAMS = pltpu.CompilerParams(use_tc_tiling_on_sc=False)

NCHUNK_D = E // CH
FULL_D = NCHUNK_D // NW
REM_D = NCHUNK_D - FULL_D * NW


def _deg_body(dst_hbm, zeros_hbm, ones_hbm, degp_hbm, dstv, onesv, deg_sh):
    cid = lax.axis_index("c")
    sid = lax.axis_index("s")
    wid = cid * NS + sid

    @pl.when(sid == 0)
    def _():
        pltpu.sync_copy(zeros_hbm, deg_sh)

    pltpu.sync_copy(ones_hbm, onesv)
    plsc.subcore_barrier()

    def do_chunk(ch):
        pltpu.sync_copy(dst_hbm.at[pl.ds(ch * CH, CH)], dstv)
        pltpu.sync_copy(onesv, deg_sh.at[dstv], add=True)

    def step(i, carry):
        do_chunk(i * NW + wid)
        return carry

    lax.fori_loop(0, FULL_D, step, 0)

    @pl.when(wid < REM_D)
    def _():
        do_chunk(FULL_D * NW + wid)

    plsc.subcore_barrier()

    @pl.when(sid == 0)
    def _():
        pltpu.sync_copy(deg_sh, degp_hbm.at[cid])


def _deg_call(dst, zeros1, ones1):
    return pl.kernel(
        _deg_body,
        out_type=jax.ShapeDtypeStruct((NC, N), jnp.float32),
        mesh=_mesh(),
        compiler_params=_SC_PARAMS,
        scratch_types=[
            pltpu.VMEM((CH,), jnp.int32),
            pltpu.VMEM((CH,), jnp.float32),
            pltpu.VMEM_SHARED((N,), jnp.float32),
        ],
    )(dst, zeros1, ones1)


def _agg_body(hs_hbm, src2_hbm, dst2_hbm, zeros_hbm, out_hbm,
              sidx, didx, buf_a, buf_b, acc_sh, sg_a, sg_b):
    cid = lax.axis_index("c")
    sid = lax.axis_index("s")
    hsv = hs_hbm.at[cid]

    pltpu.sync_copy(src2_hbm.at[pl.ds(sid * CPT, CPT)], sidx)
    pltpu.sync_copy(dst2_hbm.at[pl.ds(sid * CPT, CPT)], didx)

    @pl.when(sid == 0)
    def _():
        pltpu.sync_copy(zeros_hbm, acc_sh)

    plsc.subcore_barrier()

    def gathers(buf, sem, grp):
        return [
            pltpu.async_copy(hsv.at[sidx.at[grp * K + j]], buf.at[j], sem)
            for j in range(K)
        ]

    def scatters(buf, grp):
        for j in range(K):
            pltpu.sync_copy(buf.at[j], acc_sh.at[didx.at[grp * K + j]], add=True)

    def body(t, carry):
        ga = gathers(buf_a, sg_a, 2 * t)
        gb = gathers(buf_b, sg_b, 2 * t + 1)
        for d in ga:
            d.wait()
        scatters(buf_a, 2 * t)
        for d in gb:
            d.wait()
        scatters(buf_b, 2 * t + 1)
        return carry

    lax.fori_loop(0, T // 2, body, 0)

    plsc.subcore_barrier()

    @pl.when(sid == 0)
    def _():
        pltpu.sync_copy(acc_sh, out_hbm.at[cid])


def _agg_call(hs2, src2, dst2, zeros2):
    return pl.kernel(
        _agg_body,
        out_type=jax.ShapeDtypeStruct((NC, N2, HH), jnp.float32),
        mesh=_mesh(),
        compiler_params=_SC_PARAMS,
        scratch_types=[
            pltpu.VMEM((CPT, CH), jnp.int32),
            pltpu.VMEM((CPT, CH), jnp.int32),
            pltpu.VMEM((K, CH, HH), jnp.float32),
            pltpu.VMEM((K, CH, HH), jnp.float32),
            pltpu.VMEM_SHARED((N2, HH), jnp.float32),
            pltpu.SemaphoreType.DMA,
            pltpu.SemaphoreType.DMA,
        ],
    )(hs2, src2, dst2, zeros2)


def _prep_body(degpt_ref, x_ref, w1_ref, dis2d_ref, hs_ref):
    s = degpt_ref[...]
    deg = s[:, 0:1] + s[:, 1:2] + 1.0
    dis = lax.rsqrt(deg)
    d2 = jnp.broadcast_to(dis, (N, H))
    zpad = jnp.zeros((NPAD, H), jnp.float32)
    dis2d_ref[pl.ds(0, N), :] = d2
    dis2d_ref[pl.ds(N, NPAD), :] = zpad
    mm = d2 * jnp.dot(x_ref[...], w1_ref[...], preferred_element_type=jnp.float32)
    hs_ref[0, pl.ds(0, N), :] = mm[:, :HH]
    hs_ref[1, pl.ds(0, N), :] = mm[:, HH:]
    hs_ref[0, pl.ds(N, NPAD), :] = zpad[:, :HH]
    hs_ref[1, pl.ds(N, NPAD), :] = zpad[:, :HH]


def _prep_call(degp_t, x, w1):
    return pl.pallas_call(
        _prep_body,
        out_shape=(
            jax.ShapeDtypeStruct((N2, H), jnp.float32),
            jax.ShapeDtypeStruct((NC, N2, HH), jnp.float32),
        ),
    )(degp_t, x, w1)


def _mid_body(pp_ref, hs_ref, d2_ref, w_ref, b_ref, out_ref):
    agg = pp_ref[...] + hs_ref[...]
    aggf = jnp.concatenate([agg[0], agg[1]], axis=1)
    d2 = d2_ref[...]
    h = jnp.maximum(d2 * aggf + b_ref[...], 0.0)
    res = d2 * jnp.dot(h, w_ref[...], preferred_element_type=jnp.float32)
    out_ref[0, :, :] = res[:, :HH]
    out_ref[1, :, :] = res[:, HH:]


def _mid_call(pp, hs2, dis2d, w_next, b):
    return pl.pallas_call(
        _mid_body,
        out_shape=jax.ShapeDtypeStruct((NC, N2, HH), jnp.float32),
    )(pp, hs2, dis2d, w_next, b)


def _final_body(pp_ref, hs_ref, d2_ref, b3_ref, batch_ref, wc1_ref, bc1_ref,
                wc2_ref, bc2_ref, out_ref):
    agg = pp_ref[...] + hs_ref[...]
    aggf = jnp.concatenate([agg[0], agg[1]], axis=1)
    h = jnp.maximum(d2_ref[...] * aggf + b3_ref[...], 0.0)
    hn = h[:N, :]
    bb = batch_ref[...]
    gi = lax.broadcasted_iota(jnp.int32, (G, N), 0)
    oh = jnp.where(gi == bb, 1.0, 0.0)
    sums = jnp.dot(oh, hn, preferred_element_type=jnp.float32)
    cnts = jnp.sum(oh, axis=1, keepdims=True)
    g = sums / jnp.maximum(cnts, 1.0)
    a = jnp.maximum(
        jnp.dot(g, wc1_ref[...], preferred_element_type=jnp.float32) + bc1_ref[...],
        0.0,
    )
    out_ref[...] = (
        jnp.dot(a, wc2_ref[...], preferred_element_type=jnp.float32) + bc2_ref[...]
    )


def _final_call(pp, hs2, dis2d, b3, batch2d, wc1, bc1, wc2, bc2):
    return pl.pallas_call(
        _final_body,
        out_shape=jax.ShapeDtypeStruct((G, C), jnp.float32),
    )(pp, hs2, dis2d, b3, batch2d, wc1, bc1, wc2, bc2)


@jax.jit
def kernel(x, edge_index, batch, W1, b1, W2, b2, W3, b3, Wc1, bc1, Wc2, bc2):
    src = edge_index[0]
    dst = edge_index[1]
    pad = N + (jnp.arange(E2 - E, dtype=jnp.int32) % NPAD)
    src2 = jnp.concatenate([src, pad]).reshape(NCHUNK, CH)
    dst2 = jnp.concatenate([dst, pad]).reshape(NCHUNK, CH)

    zeros1 = jnp.zeros((N,), jnp.float32)
    ones1 = jnp.ones((CH,), jnp.float32)
    zeros2 = jnp.zeros((N2, HH), jnp.float32)

    degp = _deg_call(dst, zeros1, ones1)
    dis2d, hs2 = _prep_call(degp.T, x, W1)

    for (w_next, b) in ((W2, b1), (W3, b2)):
        pp = _agg_call(hs2, src2, dst2, zeros2)
        hs2 = _mid_call(pp, hs2, dis2d, w_next, b.reshape(1, H))

    pp = _agg_call(hs2, src2, dst2, zeros2)
    out = _final_call(
        pp, hs2, dis2d, b3.reshape(1, H), batch.reshape(1, N).astype(jnp.int32),
        Wc1, bc1.reshape(1, H // 2), Wc2, bc2.reshape(1, C),
    )
    return out


# speedup vs baseline: 23.6703x; 1.1349x over previous
import jax
import jax.numpy as jnp
from jax import lax
from jax.experimental import pallas as pl
from jax.experimental.pallas import tpu as pltpu
from jax.experimental.pallas import tpu_sc as plsc

N = 10000
E = 320000
F = 128
H = 64
HH = H // 2
G = 128
C = 10

NC = 2
NS = 16
NW = NC * NS
CH = 128

CPT = 160
K = 5
T = CPT // K
NCHUNK = NS * CPT
E2 = NCHUNK * CH
NPAD = 8
N2 = N + NPAD

CPT_D = NCHUNK // NW
KD = 8

_mesh = lambda: plsc.VectorSubcoreMesh(core_axis_name="c", subcore_axis_name="s")
_SC_PARAMS = pltpu.CompilerParams(use_tc_tiling_on_sc=False)


def _deg_body(dst2_hbm, zeros_hbm, ones_hbm, degp_hbm, didx, onesv, deg_sh, ss):
    cid = lax.axis_index("c")
    sid = lax.axis_index("s")
    wid = cid * NS + sid

    pltpu.sync_copy(dst2_hbm.at[pl.ds(wid * CPT_D, CPT_D)], didx)
    pltpu.sync_copy(ones_hbm, onesv)

    @pl.when(sid == 0)
    def _():
        pltpu.sync_copy(zeros_hbm, deg_sh)

    plsc.subcore_barrier()

    def body(t, carry):
        sc = [
            pltpu.async_copy(onesv, deg_sh.at[didx.at[t * KD + j]], ss, add=True)
            for j in range(KD)
        ]
        for d in sc:
            d.wait()
        return carry

    lax.fori_loop(0, CPT_D // KD, body, 0)

    plsc.subcore_barrier()

    @pl.when(sid == 0)
    def _():
        pltpu.sync_copy(deg_sh.at[pl.ds(0, N)], degp_hbm.at[cid])


def _deg_call(dst2, zeros1, ones1):
    return pl.kernel(
        _deg_body,
        out_type=jax.ShapeDtypeStruct((NC, N), jnp.float32),
        mesh=_mesh(),
        compiler_params=_SC_PARAMS,
        scratch_types=[
            pltpu.VMEM((CPT_D, CH), jnp.int32),
            pltpu.VMEM((CH,), jnp.float32),
            pltpu.VMEM_SHARED((N2,), jnp.float32),
            pltpu.SemaphoreType.DMA,
        ],
    )(dst2, zeros1, ones1)


def _make_agg_body(async_scatter):
    def _agg_body(hs_hbm, src2_hbm, dst2_hbm, zeros_hbm, out_hbm,
                  sidx, didx, buf_a, buf_b, acc_sh, sg_a, sg_b, ss):
        cid = lax.axis_index("c")
        sid = lax.axis_index("s")
        hsv = hs_hbm.at[cid]

        pltpu.sync_copy(src2_hbm.at[pl.ds(sid * CPT, CPT)], sidx)
        pltpu.sync_copy(dst2_hbm.at[pl.ds(sid * CPT, CPT)], didx)

        @pl.when(sid == 0)
        def _():
            pltpu.sync_copy(zeros_hbm, acc_sh)

        plsc.subcore_barrier()

        def gathers(buf, sem, grp):
            return [
                pltpu.async_copy(hsv.at[sidx.at[grp * K + j]], buf.at[j], sem)
                for j in range(K)
            ]

        def scatters(buf, grp):
            if async_scatter:
                return [
                    pltpu.async_copy(buf.at[j], acc_sh.at[didx.at[grp * K + j]],
                                     ss, add=True)
                    for j in range(K)
                ]
            for j in range(K):
                pltpu.sync_copy(buf.at[j], acc_sh.at[didx.at[grp * K + j]],
                                add=True)
            return []

        def body(t, carry):
            ga = gathers(buf_a, sg_a, 2 * t)
            gb = gathers(buf_b, sg_b, 2 * t + 1)
            for d in ga:
                d.wait()
            sa = scatters(buf_a, 2 * t)
            for d in gb:
                d.wait()
            sb = scatters(buf_b, 2 * t + 1)
            for d in sa:
                d.wait()
            for d in sb:
                d.wait()
            return carry

        lax.fori_loop(0, T // 2, body, 0)

        plsc.subcore_barrier()

        @pl.when(sid == 0)
        def _():
            pltpu.sync_copy(acc_sh, out_hbm.at[cid])

    return _agg_body


_agg_async = _make_agg_body(True)
_agg_sync = _make_agg_body(False)


def _agg_call(hs2, src2, dst2, zeros2, async_scatter):
    return pl.kernel(
        _agg_async if async_scatter else _agg_sync,
        out_type=jax.ShapeDtypeStruct((NC, N2, HH), jnp.float32),
        mesh=_mesh(),
        compiler_params=_SC_PARAMS,
        scratch_types=[
            pltpu.VMEM((CPT, CH), jnp.int32),
            pltpu.VMEM((CPT, CH), jnp.int32),
            pltpu.VMEM((K, CH, HH), jnp.float32),
            pltpu.VMEM((K, CH, HH), jnp.float32),
            pltpu.VMEM_SHARED((N2, HH), jnp.float32),
            pltpu.SemaphoreType.DMA,
            pltpu.SemaphoreType.DMA,
            pltpu.SemaphoreType.DMA,
        ],
    )(hs2, src2, dst2, zeros2)


def _mm_body(x_ref, w1_ref, mm_ref):
    mm_ref[...] = jnp.dot(x_ref[...], w1_ref[...],
                          preferred_element_type=jnp.float32)


def _mm_call(x, w1):
    return pl.pallas_call(
        _mm_body,
        out_shape=jax.ShapeDtypeStruct((N, H), jnp.float32),
    )(x, w1)


def _scale_body(degpt_ref, mm_ref, dis2d_ref, hs_ref):
    s = degpt_ref[...]
    deg = s[:, 0:1] + s[:, 1:2] + 1.0
    dis = lax.rsqrt(deg)
    d2 = jnp.broadcast_to(dis, (N, H))
    zpad = jnp.zeros((NPAD, H), jnp.float32)
    dis2d_ref[pl.ds(0, N), :] = d2
    dis2d_ref[pl.ds(N, NPAD), :] = zpad
    mm = d2 * mm_ref[...]
    hs_ref[0, pl.ds(0, N), :] = mm[:, :HH]
    hs_ref[1, pl.ds(0, N), :] = mm[:, HH:]
    hs_ref[0, pl.ds(N, NPAD), :] = zpad[:, :HH]
    hs_ref[1, pl.ds(N, NPAD), :] = zpad[:, :HH]


def _scale_call(degp_t, mm):
    return pl.pallas_call(
        _scale_body,
        out_shape=(
            jax.ShapeDtypeStruct((N2, H), jnp.float32),
            jax.ShapeDtypeStruct((NC, N2, HH), jnp.float32),
        ),
    )(degp_t, mm)


def _mid_body(pp_ref, hs_ref, d2_ref, w_ref, b_ref, out_ref):
    agg = pp_ref[...] + hs_ref[...]
    aggf = jnp.concatenate([agg[0], agg[1]], axis=1)
    d2 = d2_ref[...]
    h = jnp.maximum(d2 * aggf + b_ref[...], 0.0)
    res = d2 * jnp.dot(h, w_ref[...], preferred_element_type=jnp.float32)
    out_ref[0, :, :] = res[:, :HH]
    out_ref[1, :, :] = res[:, HH:]


def _mid_call(pp, hs2, dis2d, w_next, b):
    return pl.pallas_call(
        _mid_body,
        out_shape=jax.ShapeDtypeStruct((NC, N2, HH), jnp.float32),
    )(pp, hs2, dis2d, w_next, b)


def _final_body(pp_ref, hs_ref, d2_ref, b3_ref, batch_ref, wc1_ref, bc1_ref,
                wc2_ref, bc2_ref, out_ref):
    agg = pp_ref[...] + hs_ref[...]
    aggf = jnp.concatenate([agg[0], agg[1]], axis=1)
    h = jnp.maximum(d2_ref[...] * aggf + b3_ref[...], 0.0)
    hn = h[:N, :]
    bb = batch_ref[...]
    gi = lax.broadcasted_iota(jnp.int32, (G, N), 0)
    oh = jnp.where(gi == bb, 1.0, 0.0)
    sums = jnp.dot(oh, hn, preferred_element_type=jnp.float32)
    cnts = jnp.sum(oh, axis=1, keepdims=True)
    g = sums / jnp.maximum(cnts, 1.0)
    a = jnp.maximum(
        jnp.dot(g, wc1_ref[...], preferred_element_type=jnp.float32) + bc1_ref[...],
        0.0,
    )
    out_ref[...] = (
        jnp.dot(a, wc2_ref[...], preferred_element_type=jnp.float32) + bc2_ref[...]
    )


def _final_call(pp, hs2, dis2d, b3, batch2d, wc1, bc1, wc2, bc2):
    return pl.pallas_call(
        _final_body,
        out_shape=jax.ShapeDtypeStruct((G, C), jnp.float32),
    )(pp, hs2, dis2d, b3, batch2d, wc1, bc1, wc2, bc2)


@jax.jit
def kernel(x, edge_index, batch, W1, b1, W2, b2, W3, b3, Wc1, bc1, Wc2, bc2):
    src = edge_index[0]
    dst = edge_index[1]
    pad = N + (jnp.arange(E2 - E, dtype=jnp.int32) % NPAD)
    src2 = jnp.concatenate([src, pad]).reshape(NCHUNK, CH)
    dst2 = jnp.concatenate([dst, pad]).reshape(NCHUNK, CH)

    zeros1 = jnp.zeros((N2,), jnp.float32)
    ones1 = jnp.ones((CH,), jnp.float32)
    zeros2 = jnp.zeros((N2, HH), jnp.float32)

    mm = _mm_call(x, W1)
    degp = _deg_call(dst2, zeros1, ones1)
    dis2d, hs2 = _scale_call(degp.T, mm)

    for (w_next, b) in ((W2, b1), (W3, b2)):
        pp = _agg_call(hs2, src2, dst2, zeros2, async_scatter=True)
        hs2 = _mid_call(pp, hs2, dis2d, w_next, b.reshape(1, H))

    pp = _agg_call(hs2, src2, dst2, zeros2, async_scatter=False)
    out = _final_call(
        pp, hs2, dis2d, b3.reshape(1, H), batch.reshape(1, N).astype(jnp.int32),
        Wc1, bc1.reshape(1, H // 2), Wc2, bc2.reshape(1, C),
    )
    return out


# speedup vs baseline: 26.9998x; 1.1407x over previous
import jax
import jax.numpy as jnp
from jax import lax
from jax.experimental import pallas as pl
from jax.experimental.pallas import tpu as pltpu
from jax.experimental.pallas import tpu_sc as plsc

N = 10000
E = 320000
F = 128
H = 64
HH = H // 2
G = 128
C = 10

NC = 2
NS = 16
NW = NC * NS
CH = 128

CPT = 160
K = 8
NG = 2
T = CPT // K
NCHUNK = NS * CPT
E2 = NCHUNK * CH
NPAD = 16
N2 = N + NPAD
RPT = N2 // NS

CPT_D = NCHUNK // NW
KD = 8

_mesh = lambda: plsc.VectorSubcoreMesh(core_axis_name="c", subcore_axis_name="s")
_SC_PARAMS = pltpu.CompilerParams(use_tc_tiling_on_sc=False)


def _deg_body(dst2_hbm, zeros_hbm, ones_hbm, degp_hbm, didx, onesv, deg_sh, ss):
    cid = lax.axis_index("c")
    sid = lax.axis_index("s")
    wid = cid * NS + sid

    pltpu.sync_copy(dst2_hbm.at[pl.ds(wid * CPT_D, CPT_D)], didx)
    pltpu.sync_copy(ones_hbm, onesv)

    @pl.when(sid == 0)
    def _():
        pltpu.sync_copy(zeros_hbm, deg_sh)

    plsc.subcore_barrier()

    def body(t, carry):
        sc = [
            pltpu.async_copy(onesv, deg_sh.at[didx.at[t * KD + j]], ss, add=True)
            for j in range(KD)
        ]
        for d in sc:
            d.wait()
        return carry

    lax.fori_loop(0, CPT_D // KD, body, 0)

    plsc.subcore_barrier()

    @pl.when(sid == 0)
    def _():
        pltpu.sync_copy(deg_sh.at[pl.ds(0, N)], degp_hbm.at[cid])


def _deg_call(dst2, zeros1, ones1):
    return pl.kernel(
        _deg_body,
        out_type=jax.ShapeDtypeStruct((NC, N), jnp.float32),
        mesh=_mesh(),
        compiler_params=_SC_PARAMS,
        scratch_types=[
            pltpu.VMEM((CPT_D, CH), jnp.int32),
            pltpu.VMEM((CH,), jnp.float32),
            pltpu.VMEM_SHARED((N2,), jnp.float32),
            pltpu.SemaphoreType.DMA,
        ],
    )(dst2, zeros1, ones1)


def _make_agg_body(async_scatter):
    def _agg_body(hs_hbm, src2_hbm, dst2_hbm, zeros_hbm, out_hbm,
                  sidx, didx, b0, b1, acc_sh, g0, g1, ss):
        cid = lax.axis_index("c")
        sid = lax.axis_index("s")
        hsv = hs_hbm.at[cid]
        bufs = (b0, b1)
        gsems = (g0, g1)

        pltpu.sync_copy(src2_hbm.at[pl.ds(sid * CPT, CPT)], sidx)
        pltpu.sync_copy(dst2_hbm.at[pl.ds(sid * CPT, CPT)], didx)

        @pl.when(sid == 0)
        def _():
            pltpu.sync_copy(zeros_hbm, acc_sh)

        plsc.subcore_barrier()

        def gathers(buf, sem, grp):
            return [
                pltpu.async_copy(hsv.at[sidx.at[grp * K + j]], buf.at[j], sem)
                for j in range(K)
            ]

        def scatters(buf, grp):
            if async_scatter:
                return [
                    pltpu.async_copy(buf.at[j], acc_sh.at[didx.at[grp * K + j]],
                                     ss, add=True)
                    for j in range(K)
                ]
            for j in range(K):
                pltpu.sync_copy(buf.at[j], acc_sh.at[didx.at[grp * K + j]],
                                add=True)
            return []

        def body(t, carry):
            gs = [gathers(bufs[q], gsems[q], NG * t + q) for q in range(NG)]
            pend = []
            for q in range(NG):
                for d in gs[q]:
                    d.wait()
                pend += scatters(bufs[q], NG * t + q)
            for d in pend:
                d.wait()
            return carry

        lax.fori_loop(0, T // NG, body, 0)

        plsc.subcore_barrier()

        @pl.when(sid == 0)
        def _():
            pltpu.sync_copy(acc_sh, out_hbm.at[cid])

    return _agg_body


_agg_async = _make_agg_body(True)
_agg_sync = _make_agg_body(False)


def _agg_call(hs2, src2, dst2, zeros2, async_scatter):
    return pl.kernel(
        _agg_async if async_scatter else _agg_sync,
        out_type=jax.ShapeDtypeStruct((NC, N2, HH), jnp.float32),
        mesh=_mesh(),
        compiler_params=_SC_PARAMS,
        scratch_types=[
            pltpu.VMEM((CPT, CH), jnp.int32),
            pltpu.VMEM((CPT, CH), jnp.int32),
            pltpu.VMEM((K, CH, HH), jnp.float32),
            pltpu.VMEM((K, CH, HH), jnp.float32),
            pltpu.VMEM_SHARED((N2, HH), jnp.float32),
            pltpu.SemaphoreType.DMA,
            pltpu.SemaphoreType.DMA,
            pltpu.SemaphoreType.DMA,
        ],
    )(hs2, src2, dst2, zeros2)


def _prep_body(degpt_ref, x_ref, w1_ref, dis2d_ref, hs_ref):
    s = degpt_ref[...]
    deg = s[:, 0:1] + s[:, 1:2] + 1.0
    dis = lax.rsqrt(deg)
    d2 = jnp.broadcast_to(dis, (N, H))
    zpad = jnp.zeros((NPAD, H), jnp.float32)
    dis2d_ref[pl.ds(0, N), :] = d2
    dis2d_ref[pl.ds(N, NPAD), :] = zpad
    mm = d2 * jnp.dot(x_ref[...], w1_ref[...], preferred_element_type=jnp.float32)
    hs_ref[0, pl.ds(0, N), :] = mm[:, :HH]
    hs_ref[1, pl.ds(0, N), :] = mm[:, HH:]
    hs_ref[0, pl.ds(N, NPAD), :] = zpad[:, :HH]
    hs_ref[1, pl.ds(N, NPAD), :] = zpad[:, :HH]


def _prep_call(degp_t, x, w1):
    return pl.pallas_call(
        _prep_body,
        out_shape=(
            jax.ShapeDtypeStruct((N2, H), jnp.float32),
            jax.ShapeDtypeStruct((NC, N2, HH), jnp.float32),
        ),
    )(degp_t, x, w1)


def _mid_body(pp_ref, hs_ref, d2_ref, w_ref, b_ref, out_ref):
    agg = pp_ref[...] + hs_ref[...]
    aggf = jnp.concatenate([agg[0], agg[1]], axis=1)
    d2 = d2_ref[...]
    h = jnp.maximum(d2 * aggf + b_ref[...], 0.0)
    res = d2 * jnp.dot(h, w_ref[...], preferred_element_type=jnp.float32)
    out_ref[0, :, :] = res[:, :HH]
    out_ref[1, :, :] = res[:, HH:]


def _mid_call(pp, hs2, dis2d, w_next, b):
    return pl.pallas_call(
        _mid_body,
        out_shape=jax.ShapeDtypeStruct((NC, N2, HH), jnp.float32),
    )(pp, hs2, dis2d, w_next, b)


def _final_body(pp_ref, hs_ref, d2_ref, b3_ref, batch_ref, wc1_ref, bc1_ref,
                wc2_ref, bc2_ref, out_ref):
    agg = pp_ref[...] + hs_ref[...]
    aggf = jnp.concatenate([agg[0], agg[1]], axis=1)
    h = jnp.maximum(d2_ref[...] * aggf + b3_ref[...], 0.0)
    hn = h[:N, :]
    bb = batch_ref[...]
    gi = lax.broadcasted_iota(jnp.int32, (G, N), 0)
    oh = jnp.where(gi == bb, 1.0, 0.0)
    sums = jnp.dot(oh, hn, preferred_element_type=jnp.float32)
    cnts = jnp.sum(oh, axis=1, keepdims=True)
    g = sums / jnp.maximum(cnts, 1.0)
    a = jnp.maximum(
        jnp.dot(g, wc1_ref[...], preferred_element_type=jnp.float32) + bc1_ref[...],
        0.0,
    )
    out_ref[...] = (
        jnp.dot(a, wc2_ref[...], preferred_element_type=jnp.float32) + bc2_ref[...]
    )


def _final_call(pp, hs2, dis2d, b3, batch2d, wc1, bc1, wc2, bc2):
    return pl.pallas_call(
        _final_body,
        out_shape=jax.ShapeDtypeStruct((G, C), jnp.float32),
    )(pp, hs2, dis2d, b3, batch2d, wc1, bc1, wc2, bc2)


@jax.jit
def kernel(x, edge_index, batch, W1, b1, W2, b2, W3, b3, Wc1, bc1, Wc2, bc2):
    src = edge_index[0]
    dst = edge_index[1]
    pad = N + (jnp.arange(E2 - E, dtype=jnp.int32) % NPAD)
    src2 = jnp.concatenate([src, pad]).reshape(NCHUNK, CH)
    dst2 = jnp.concatenate([dst, pad]).reshape(NCHUNK, CH)

    zeros1 = jnp.zeros((N2,), jnp.float32)
    ones1 = jnp.ones((CH,), jnp.float32)
    zeros2 = jnp.zeros((N2, HH), jnp.float32)

    degp = _deg_call(dst2, zeros1, ones1)
    dis2d, hs2 = _prep_call(degp.T, x, W1)

    for (w_next, b) in ((W2, b1), (W3, b2)):
        pp = _agg_call(hs2, src2, dst2, zeros2, async_scatter=False)
        hs2 = _mid_call(pp, hs2, dis2d, w_next, b.reshape(1, H))

    pp = _agg_call(hs2, src2, dst2, zeros2, async_scatter=False)
    out = _final_call(
        pp, hs2, dis2d, b3.reshape(1, H), batch.reshape(1, N).astype(jnp.int32),
        Wc1, bc1.reshape(1, H // 2), Wc2, bc2.reshape(1, C),
    )
    return out


# speedup vs baseline: 27.0278x; 1.0010x over previous
import jax
import jax.numpy as jnp
from jax import lax
from jax.experimental import pallas as pl
from jax.experimental.pallas import tpu as pltpu
from jax.experimental.pallas import tpu_sc as plsc

N = 10000
E = 320000
F = 128
H = 64
HH = H // 2
G = 128
C = 10

NC = 2
NS = 16
NW = NC * NS
CH = 128

CPT = 160
K = 8
NG = 2
T = CPT // K
NCHUNK = NS * CPT
E2 = NCHUNK * CH
NPAD = 16
N2 = N + NPAD
RPT = N2 // NS

CPT_D = NCHUNK // NW
KD = 8

_mesh = lambda: plsc.VectorSubcoreMesh(core_axis_name="c", subcore_axis_name="s")
_SC_PARAMS = pltpu.CompilerParams(use_tc_tiling_on_sc=False,
                                  skip_device_barrier=True)


def _deg_body(dst2_hbm, zeros_hbm, ones_hbm, degp_hbm, didx, onesv, deg_sh, ss):
    cid = lax.axis_index("c")
    sid = lax.axis_index("s")
    wid = cid * NS + sid

    pltpu.sync_copy(dst2_hbm.at[pl.ds(wid * CPT_D, CPT_D)], didx)
    pltpu.sync_copy(ones_hbm, onesv)

    @pl.when(sid == 0)
    def _():
        pltpu.sync_copy(zeros_hbm, deg_sh)

    plsc.subcore_barrier()

    def body(t, carry):
        sc = [
            pltpu.async_copy(onesv, deg_sh.at[didx.at[t * KD + j]], ss, add=True)
            for j in range(KD)
        ]
        for d in sc:
            d.wait()
        return carry

    lax.fori_loop(0, CPT_D // KD, body, 0)

    plsc.subcore_barrier()

    @pl.when(sid == 0)
    def _():
        pltpu.sync_copy(deg_sh.at[pl.ds(0, N)], degp_hbm.at[cid])


def _deg_call(dst2, zeros1, ones1):
    return pl.kernel(
        _deg_body,
        out_type=jax.ShapeDtypeStruct((NC, N), jnp.float32),
        mesh=_mesh(),
        compiler_params=_SC_PARAMS,
        scratch_types=[
            pltpu.VMEM((CPT_D, CH), jnp.int32),
            pltpu.VMEM((CH,), jnp.float32),
            pltpu.VMEM_SHARED((N2,), jnp.float32),
            pltpu.SemaphoreType.DMA,
        ],
    )(dst2, zeros1, ones1)


def _make_agg_body(async_scatter):
    def _agg_body(hs_hbm, src2_hbm, dst2_hbm, zeros_hbm, out_hbm,
                  sidx, didx, b0, b1, acc_sh, g0, g1, ss):
        cid = lax.axis_index("c")
        sid = lax.axis_index("s")
        hsv = hs_hbm.at[cid]
        bufs = (b0, b1)
        gsems = (g0, g1)

        pltpu.sync_copy(src2_hbm.at[pl.ds(sid * CPT, CPT)], sidx)
        pltpu.sync_copy(dst2_hbm.at[pl.ds(sid * CPT, CPT)], didx)

        @pl.when(sid == 0)
        def _():
            pltpu.sync_copy(zeros_hbm, acc_sh)

        plsc.subcore_barrier()

        def gathers(buf, sem, grp):
            return [
                pltpu.async_copy(hsv.at[sidx.at[grp * K + j]], buf.at[j], sem)
                for j in range(K)
            ]

        def scatters(buf, grp):
            if async_scatter:
                return [
                    pltpu.async_copy(buf.at[j], acc_sh.at[didx.at[grp * K + j]],
                                     ss, add=True)
                    for j in range(K)
                ]
            for j in range(K):
                pltpu.sync_copy(buf.at[j], acc_sh.at[didx.at[grp * K + j]],
                                add=True)
            return []

        def body(t, carry):
            gs = [gathers(bufs[q], gsems[q], NG * t + q) for q in range(NG)]
            pend = []
            for q in range(NG):
                for d in gs[q]:
                    d.wait()
                pend += scatters(bufs[q], NG * t + q)
            for d in pend:
                d.wait()
            return carry

        lax.fori_loop(0, T // NG, body, 0)

        plsc.subcore_barrier()

        @pl.when(sid == 0)
        def _():
            pltpu.sync_copy(acc_sh, out_hbm.at[cid])

    return _agg_body


_agg_async = _make_agg_body(True)
_agg_sync = _make_agg_body(False)


def _agg_call(hs2, src2, dst2, zeros2, async_scatter):
    return pl.kernel(
        _agg_async if async_scatter else _agg_sync,
        out_type=jax.ShapeDtypeStruct((NC, N2, HH), jnp.float32),
        mesh=_mesh(),
        compiler_params=_SC_PARAMS,
        scratch_types=[
            pltpu.VMEM((CPT, CH), jnp.int32),
            pltpu.VMEM((CPT, CH), jnp.int32),
            pltpu.VMEM((K, CH, HH), jnp.float32),
            pltpu.VMEM((K, CH, HH), jnp.float32),
            pltpu.VMEM_SHARED((N2, HH), jnp.float32),
            pltpu.SemaphoreType.DMA,
            pltpu.SemaphoreType.DMA,
            pltpu.SemaphoreType.DMA,
        ],
    )(hs2, src2, dst2, zeros2)


def _prep_body(degpt_ref, x_ref, w1_ref, dis2d_ref, hs_ref):
    s = degpt_ref[...]
    deg = s[:, 0:1] + s[:, 1:2] + 1.0
    dis = lax.rsqrt(deg)
    d2 = jnp.broadcast_to(dis, (N, H))
    zpad = jnp.zeros((NPAD, H), jnp.float32)
    dis2d_ref[pl.ds(0, N), :] = d2
    dis2d_ref[pl.ds(N, NPAD), :] = zpad
    mm = d2 * jnp.dot(x_ref[...], w1_ref[...], preferred_element_type=jnp.float32)
    hs_ref[0, pl.ds(0, N), :] = mm[:, :HH]
    hs_ref[1, pl.ds(0, N), :] = mm[:, HH:]
    hs_ref[0, pl.ds(N, NPAD), :] = zpad[:, :HH]
    hs_ref[1, pl.ds(N, NPAD), :] = zpad[:, :HH]


def _prep_call(degp_t, x, w1):
    return pl.pallas_call(
        _prep_body,
        out_shape=(
            jax.ShapeDtypeStruct((N2, H), jnp.float32),
            jax.ShapeDtypeStruct((NC, N2, HH), jnp.float32),
        ),
    )(degp_t, x, w1)


def _mid_body(pp_ref, hs_ref, d2_ref, w_ref, b_ref, out_ref):
    agg = pp_ref[...] + hs_ref[...]
    aggf = jnp.concatenate([agg[0], agg[1]], axis=1)
    d2 = d2_ref[...]
    h = jnp.maximum(d2 * aggf + b_ref[...], 0.0)
    res = d2 * jnp.dot(h, w_ref[...], preferred_element_type=jnp.float32)
    out_ref[0, :, :] = res[:, :HH]
    out_ref[1, :, :] = res[:, HH:]


def _mid_call(pp, hs2, dis2d, w_next, b):
    return pl.pallas_call(
        _mid_body,
        out_shape=jax.ShapeDtypeStruct((NC, N2, HH), jnp.float32),
    )(pp, hs2, dis2d, w_next, b)


def _final_body(pp_ref, hs_ref, d2_ref, b3_ref, batch_ref, wc1_ref, bc1_ref,
                wc2_ref, bc2_ref, out_ref):
    agg = pp_ref[...] + hs_ref[...]
    aggf = jnp.concatenate([agg[0], agg[1]], axis=1)
    h = jnp.maximum(d2_ref[...] * aggf + b3_ref[...], 0.0)
    hn = h[:N, :]
    bb = batch_ref[...]
    gi = lax.broadcasted_iota(jnp.int32, (G, N), 0)
    oh = jnp.where(gi == bb, 1.0, 0.0)
    sums = jnp.dot(oh, hn, preferred_element_type=jnp.float32)
    cnts = jnp.sum(oh, axis=1, keepdims=True)
    g = sums / jnp.maximum(cnts, 1.0)
    a = jnp.maximum(
        jnp.dot(g, wc1_ref[...], preferred_element_type=jnp.float32) + bc1_ref[...],
        0.0,
    )
    out_ref[...] = (
        jnp.dot(a, wc2_ref[...], preferred_element_type=jnp.float32) + bc2_ref[...]
    )


def _final_call(pp, hs2, dis2d, b3, batch2d, wc1, bc1, wc2, bc2):
    return pl.pallas_call(
        _final_body,
        out_shape=jax.ShapeDtypeStruct((G, C), jnp.float32),
    )(pp, hs2, dis2d, b3, batch2d, wc1, bc1, wc2, bc2)


@jax.jit
def kernel(x, edge_index, batch, W1, b1, W2, b2, W3, b3, Wc1, bc1, Wc2, bc2):
    src = edge_index[0]
    dst = edge_index[1]
    pad = N + (jnp.arange(E2 - E, dtype=jnp.int32) % NPAD)
    src2 = jnp.concatenate([src, pad]).reshape(NCHUNK, CH)
    dst2 = jnp.concatenate([dst, pad]).reshape(NCHUNK, CH)

    zeros1 = jnp.zeros((N2,), jnp.float32)
    ones1 = jnp.ones((CH,), jnp.float32)
    zeros2 = jnp.zeros((N2, HH), jnp.float32)

    degp = _deg_call(dst2, zeros1, ones1)
    dis2d, hs2 = _prep_call(degp.T, x, W1)

    for (w_next, b) in ((W2, b1), (W3, b2)):
        pp = _agg_call(hs2, src2, dst2, zeros2, async_scatter=False)
        hs2 = _mid_call(pp, hs2, dis2d, w_next, b.reshape(1, H))

    pp = _agg_call(hs2, src2, dst2, zeros2, async_scatter=False)
    out = _final_call(
        pp, hs2, dis2d, b3.reshape(1, H), batch.reshape(1, N).astype(jnp.int32),
        Wc1, bc1.reshape(1, H // 2), Wc2, bc2.reshape(1, C),
    )
    return out
